# baseline (speedup 1.0000x reference)
import functools

import jax
import jax.numpy as jnp
from jax import lax
from jax.experimental import pallas as pl
from jax.experimental.pallas import tpu as pltpu
from jax.experimental.pallas import tpu_sc as plsc

_LANES = 16
_NUM_WORKERS = 32
_T = 256


@functools.partial(jax.jit, static_argnums=(10, 11, 12))
def _sc_embed(x, ted, tsumo, ri, called, table, tb, rb, tsb, clb, N, D, V):
    KD = D // _LANES
    tok_per_w = N // _NUM_WORKERS
    chunks = tok_per_w // _T
    assert chunks % 2 == 0

    mesh = plsc.VectorSubcoreMesh(core_axis_name="c", subcore_axis_name="s")

    @functools.partial(
        pl.kernel,
        out_type=jax.ShapeDtypeStruct((N, D), jnp.float32),
        mesh=mesh,
        scratch_types=[
            pltpu.VMEM((V * D,), jnp.float32),
            pltpu.VMEM((4, D), jnp.float32),
            pltpu.VMEM((2, _T), jnp.int32),
            pltpu.VMEM((2, 4, _T), jnp.float32),
            pltpu.VMEM((2, _T, D), jnp.float32),
            pltpu.SemaphoreType.DMA((2,)),
            pltpu.SemaphoreType.DMA((2,)),
        ],
    )
    def k(x_hbm, ted_hbm, tsumo_hbm, ri_hbm, called_hbm, table_hbm,
          tb_hbm, rb_hbm, tsb_hbm, clb_hbm, out_hbm,
          table_v, bias_v, x_v, sc_v, out_v, sem_in, sem_out):
        wid = lax.axis_index("s") * 2 + lax.axis_index("c")

        pltpu.sync_copy(table_hbm, table_v)
        pltpu.sync_copy(tb_hbm, bias_v.at[0])
        pltpu.sync_copy(rb_hbm, bias_v.at[1])
        pltpu.sync_copy(tsb_hbm, bias_v.at[2])
        pltpu.sync_copy(clb_hbm, bias_v.at[3])

        def in_copies(c, b):
            tok0 = pl.multiple_of(wid * tok_per_w + c * _T, _T)
            return [
                pltpu.make_async_copy(
                    x_hbm.at[pl.ds(tok0, _T)], x_v.at[b], sem_in.at[b]),
                pltpu.make_async_copy(
                    ted_hbm.at[pl.ds(tok0, _T)], sc_v.at[b, 0], sem_in.at[b]),
                pltpu.make_async_copy(
                    tsumo_hbm.at[pl.ds(tok0, _T)], sc_v.at[b, 1], sem_in.at[b]),
                pltpu.make_async_copy(
                    ri_hbm.at[pl.ds(tok0, _T)], sc_v.at[b, 2], sem_in.at[b]),
                pltpu.make_async_copy(
                    called_hbm.at[pl.ds(tok0, _T)], sc_v.at[b, 3], sem_in.at[b]),
            ]

        def out_copy(c, b):
            tok0 = pl.multiple_of(wid * tok_per_w + c * _T, _T)
            return pltpu.make_async_copy(
                out_v.at[b], out_hbm.at[pl.ds(tok0, _T)], sem_out.at[b])

        cs = [bias_v[2, pl.ds(kk * _LANES, _LANES)]
              + bias_v[3, pl.ds(kk * _LANES, _LANES)] for kk in range(KD)]

        def fold(j, _):
            base = pl.multiple_of(j * D, D)
            for kk in range(KD):
                off = base + kk * _LANES
                table_v[pl.ds(off, _LANES)] = table_v[pl.ds(off, _LANES)] + cs[kk]
            return 0
        lax.fori_loop(0, V, fold, 0)

        tbs = [bias_v[0, pl.ds(kk * _LANES, _LANES)] for kk in range(KD)]
        rbs = [bias_v[1, pl.ds(kk * _LANES, _LANES)] for kk in range(KD)]

        for cp in in_copies(0, 0):
            cp.start()

        def pair(cc, _):
            for b in range(2):
                c = cc * 2 + b

                @pl.when(c + 1 < chunks)
                def _prefetch():
                    for cp in in_copies(c + 1, 1 - b):
                        cp.start()

                for cp in in_copies(c, b):
                    cp.wait()

                @pl.when(c >= 2)
                def _drain():
                    out_copy(c - 2, b).wait()

                @plsc.parallel_loop(0, _T // _LANES, 1, unroll=8)
                def _grp(g):
                    t0 = pl.multiple_of(g * _LANES, _LANES)
                    x16 = x_v[b, pl.ds(t0, _LANES)]
                    ted16 = sc_v[b, 0, pl.ds(t0, _LANES)]
                    ri16 = sc_v[b, 2, pl.ds(t0, _LANES)]
                    st16 = (sc_v[b, 1, pl.ds(t0, _LANES)]
                            + sc_v[b, 3, pl.ds(t0, _LANES)])
                    for lane in range(_LANES):
                        base = pl.multiple_of(x16[lane] * D, D)
                        tedt = ted16[lane]
                        rit = ri16[lane]
                        st = st16[lane]
                        rows = [table_v[pl.ds(base + kk * _LANES, _LANES)]
                                for kk in range(KD)]
                        vals = [(rows[kk] + st)
                                + (tedt * tbs[kk] + rit * rbs[kk])
                                for kk in range(KD)]
                        for kk in range(KD):
                            out_v[b, t0 + lane,
                                  pl.ds(kk * _LANES, _LANES)] = vals[kk]

                out_copy(c, b).start()
            return 0
        lax.fori_loop(0, chunks // 2, pair, 0)

        out_copy(chunks - 2, 0).wait()
        out_copy(chunks - 1, 1).wait()

    return k(x, ted, tsumo, ri, called, table, tb, rb, tsb, clb)


def kernel(x, tedashi, tsumogiri, riichi, called, table,
           tedashi_bias, tsumogiri_bias, riichi_bias, called_bias):
    B, L = x.shape
    V, D = table.shape
    N = B * L
    out = _sc_embed(
        x.reshape(N).astype(jnp.int32),
        tedashi.reshape(N), tsumogiri.reshape(N),
        riichi.reshape(N), called.reshape(N),
        table.reshape(V * D),
        tedashi_bias.reshape(D), riichi_bias.reshape(D),
        tsumogiri_bias.reshape(D), called_bias.reshape(D),
        N, D, V)
    return out.reshape(B, L, D)


# speedup vs baseline: 2.1538x; 2.1538x over previous
import functools

import jax
import jax.numpy as jnp
from jax import lax
from jax.experimental import pallas as pl
from jax.experimental.pallas import tpu as pltpu
from jax.experimental.pallas import tpu_sc as plsc

_LANES = 16
_NUM_WORKERS = 32
_T = 256


@functools.partial(jax.jit, static_argnums=(10, 11, 12))
def _sc_embed(x, ted, tsumo, ri, called, table, tb, rb, tsb, clb, N, D, V):
    KD = D // _LANES
    tok_per_w = N // _NUM_WORKERS
    chunks = tok_per_w // _T
    assert chunks % 2 == 0

    mesh = plsc.VectorSubcoreMesh(core_axis_name="c", subcore_axis_name="s")

    @functools.partial(
        pl.kernel,
        out_type=jax.ShapeDtypeStruct((N, D), jnp.float32),
        mesh=mesh,
        scratch_types=[
            pltpu.VMEM((V * D,), jnp.float32),
            pltpu.VMEM((4, D), jnp.float32),
            pltpu.VMEM((2, _T), jnp.int32),
            pltpu.VMEM((2, 4, _T), jnp.float32),
            pltpu.VMEM((2, _T, D), jnp.float32),
            pltpu.SemaphoreType.DMA((2,)),
            pltpu.SemaphoreType.DMA((2,)),
        ],
    )
    def k(x_hbm, ted_hbm, tsumo_hbm, ri_hbm, called_hbm, table_hbm,
          tb_hbm, rb_hbm, tsb_hbm, clb_hbm, out_hbm,
          table_v, bias_v, x_v, sc_v, out_v, sem_in, sem_out):
        wid = lax.axis_index("s") * 2 + lax.axis_index("c")

        pltpu.sync_copy(table_hbm, table_v)
        pltpu.sync_copy(tb_hbm, bias_v.at[0])
        pltpu.sync_copy(rb_hbm, bias_v.at[1])
        pltpu.sync_copy(tsb_hbm, bias_v.at[2])
        pltpu.sync_copy(clb_hbm, bias_v.at[3])

        def in_copies(c, b):
            tok0 = pl.multiple_of(wid * tok_per_w + c * _T, _T)
            return [
                pltpu.make_async_copy(
                    x_hbm.at[pl.ds(tok0, _T)], x_v.at[b], sem_in.at[b]),
                pltpu.make_async_copy(
                    ted_hbm.at[pl.ds(tok0, _T)], sc_v.at[b, 0], sem_in.at[b]),
                pltpu.make_async_copy(
                    tsumo_hbm.at[pl.ds(tok0, _T)], sc_v.at[b, 1], sem_in.at[b]),
                pltpu.make_async_copy(
                    ri_hbm.at[pl.ds(tok0, _T)], sc_v.at[b, 2], sem_in.at[b]),
                pltpu.make_async_copy(
                    called_hbm.at[pl.ds(tok0, _T)], sc_v.at[b, 3], sem_in.at[b]),
            ]

        def out_copy(c, b):
            tok0 = pl.multiple_of(wid * tok_per_w + c * _T, _T)
            return pltpu.make_async_copy(
                out_v.at[b], out_hbm.at[pl.ds(tok0, _T)], sem_out.at[b])

        cs = [bias_v[2, pl.ds(kk * _LANES, _LANES)]
              + bias_v[3, pl.ds(kk * _LANES, _LANES)] for kk in range(KD)]

        def fold(j, _):
            base = pl.multiple_of(j * D, D)
            for kk in range(KD):
                off = base + kk * _LANES
                table_v[pl.ds(off, _LANES)] = table_v[pl.ds(off, _LANES)] + cs[kk]
            return 0
        lax.fori_loop(0, V, fold, 0)

        tbs = [bias_v[0, pl.ds(kk * _LANES, _LANES)] for kk in range(KD)]
        rbs = [bias_v[1, pl.ds(kk * _LANES, _LANES)] for kk in range(KD)]

        for cp in in_copies(0, 0):
            cp.start()

        def pair(cc, _):
            for b in range(2):
                c = cc * 2 + b

                @pl.when(c + 1 < chunks)
                def _prefetch():
                    for cp in in_copies(c + 1, 1 - b):
                        cp.start()

                for cp in in_copies(c, b):
                    cp.wait()

                @pl.when(c >= 2)
                def _drain():
                    out_copy(c - 2, b).wait()

                @plsc.parallel_loop(0, _T // _LANES, 1, unroll=4)
                def _grp(g):
                    t0 = pl.multiple_of(g * _LANES, _LANES)
                    x16 = x_v[b, pl.ds(t0, _LANES)]
                    ted16 = sc_v[b, 0, pl.ds(t0, _LANES)]
                    ri16 = sc_v[b, 2, pl.ds(t0, _LANES)]
                    st16 = (sc_v[b, 1, pl.ds(t0, _LANES)]
                            + sc_v[b, 3, pl.ds(t0, _LANES)])
                    for lane in range(_LANES):
                        base = pl.multiple_of(x16[lane] * D, D)
                        tedt = ted16[lane]
                        rit = ri16[lane]
                        st = st16[lane]
                        rows = [table_v[pl.ds(base + kk * _LANES, _LANES)]
                                for kk in range(KD)]
                        vals = [(rows[kk] + st)
                                + (tedt * tbs[kk] + rit * rbs[kk])
                                for kk in range(KD)]
                        for kk in range(KD):
                            out_v[b, t0 + lane,
                                  pl.ds(kk * _LANES, _LANES)] = vals[kk]

                out_copy(c, b).start()
            return 0
        lax.fori_loop(0, chunks // 2, pair, 0)

        out_copy(chunks - 2, 0).wait()
        out_copy(chunks - 1, 1).wait()

    return k(x, ted, tsumo, ri, called, table, tb, rb, tsb, clb)


def kernel(x, tedashi, tsumogiri, riichi, called, table,
           tedashi_bias, tsumogiri_bias, riichi_bias, called_bias):
    B, L = x.shape
    V, D = table.shape
    N = B * L
    out = _sc_embed(
        x.reshape(N).astype(jnp.int32),
        tedashi.reshape(N), tsumogiri.reshape(N),
        riichi.reshape(N), called.reshape(N),
        table.reshape(V * D),
        tedashi_bias.reshape(D), riichi_bias.reshape(D),
        tsumogiri_bias.reshape(D), called_bias.reshape(D),
        N, D, V)
    return out.reshape(B, L, D)
